# baseline (speedup 1.0000x reference)
import jax
import jax.numpy as jnp
from jax import lax
from jax.experimental import pallas as pl
from jax.experimental.pallas import tpu as pltpu
from jax.experimental.pallas import tpu_sc as plsc

NUM_CORES = 2
NUM_SUBCORES = 16
NUM_WORKERS = NUM_CORES * NUM_SUBCORES
CB = 4
NBUF = 4


def _gather_body(idx_hbm, table_hbm, out_hbm, idx_v, *scratch):
    bufs = scratch[:NBUF]
    gsems = scratch[NBUF : 2 * NBUF]
    ssems = scratch[2 * NBUF :]
    wid = lax.axis_index("s") * NUM_CORES + lax.axis_index("c")
    rows_per_w = idx_hbm.shape[1]
    base = wid * rows_per_w
    nchunks = rows_per_w // CB
    nrounds = nchunks // NBUF

    pltpu.sync_copy(idx_hbm.at[wid], idx_v)

    def gathers_start(b, c):
        for j in range(CB):
            pltpu.async_copy(table_hbm.at[idx_v.at[c * CB + j]], bufs[b].at[j], gsems[b])

    def gathers_wait(b, c):
        for j in range(CB):
            pltpu.make_async_copy(
                table_hbm.at[idx_v.at[c * CB + j]], bufs[b].at[j], gsems[b]
            ).wait()

    def store_start(b, c):
        pltpu.async_copy(bufs[b], out_hbm.at[pl.ds(base + c * CB, CB)], ssems[b])

    def store_wait(b, c):
        pltpu.make_async_copy(bufs[b], out_hbm.at[pl.ds(base + c * CB, CB)], ssems[b]).wait()

    for b in range(NBUF):
        gathers_start(b, b)

    def round_body(r, carry):
        cbase = NBUF * r
        for b in range(NBUF):
            gathers_wait(b, cbase + b)
            store_start(b, cbase + b)

        @pl.when(r < nrounds - 1)
        def _prefetch():
            for b in range(NBUF):
                store_wait(b, cbase + b)
                gathers_start(b, cbase + NBUF + b)

        return carry

    lax.fori_loop(0, nrounds, round_body, 0)
    for b in range(NBUF):
        store_wait(b, nchunks - NBUF + b)


NSPLIT = 4


def kernel(encoding_indices, table):
    B, S = encoding_indices.shape
    V, D = table.shape
    bc = B // NSPLIT
    rows_per_w = bc // NUM_WORKERS
    mesh = plsc.VectorSubcoreMesh(core_axis_name="c", subcore_axis_name="s")
    call = pl.kernel(
        _gather_body,
        out_type=jax.ShapeDtypeStruct((bc, S, D), jnp.float32),
        mesh=mesh,
        scratch_types=(
            [pltpu.VMEM((rows_per_w, S), jnp.int32)]
            + [pltpu.VMEM((CB, S, D), jnp.float32) for _ in range(NBUF)]
            + [pltpu.SemaphoreType.DMA for _ in range(2 * NBUF)]
        ),
    )
    idx = encoding_indices.reshape(NSPLIT, NUM_WORKERS, rows_per_w, S).astype(jnp.int32)
    outs = [call(idx[k], table) for k in range(NSPLIT)]
    return jnp.concatenate(outs, axis=0)


# speedup vs baseline: 1.7991x; 1.7991x over previous
import jax
import jax.numpy as jnp
from jax import lax
from jax.experimental import pallas as pl
from jax.experimental.pallas import tpu as pltpu
from jax.experimental.pallas import tpu_sc as plsc

NUM_CORES = 2
NUM_SUBCORES = 16
NUM_WORKERS = NUM_CORES * NUM_SUBCORES
CB = 4
NBUF = 4


def _gather_body(idx_hbm, table_hbm, out_hbm, idx_v, *scratch):
    bufs = scratch[:NBUF]
    gsems = scratch[NBUF : 2 * NBUF]
    ssems = scratch[2 * NBUF :]
    wid = lax.axis_index("s") * NUM_CORES + lax.axis_index("c")
    rows_per_w = idx_hbm.shape[1]
    base = wid * rows_per_w
    nchunks = rows_per_w // CB
    nrounds = nchunks // NBUF

    pltpu.sync_copy(idx_hbm.at[wid], idx_v)

    def gathers_start(b, c):
        for j in range(CB):
            pltpu.async_copy(table_hbm.at[idx_v.at[c * CB + j]], bufs[b].at[j], gsems[b])

    def gathers_wait(b, c):
        for j in range(CB):
            pltpu.make_async_copy(
                table_hbm.at[idx_v.at[c * CB + j]], bufs[b].at[j], gsems[b]
            ).wait()

    def store_start(b, c):
        pltpu.async_copy(bufs[b], out_hbm.at[pl.ds(base + c * CB, CB)], ssems[b])

    def store_wait(b, c):
        pltpu.make_async_copy(bufs[b], out_hbm.at[pl.ds(base + c * CB, CB)], ssems[b]).wait()

    for b in range(NBUF):
        gathers_start(b, b)

    def round_body(r, carry):
        cbase = NBUF * r
        for b in range(NBUF):
            gathers_wait(b, cbase + b)
            store_start(b, cbase + b)

        @pl.when(r < nrounds - 1)
        def _prefetch():
            for b in range(NBUF):
                store_wait(b, cbase + b)
                gathers_start(b, cbase + NBUF + b)

        return carry

    lax.fori_loop(0, nrounds, round_body, 0)
    for b in range(NBUF):
        store_wait(b, nchunks - NBUF + b)


def kernel(encoding_indices, table):
    B, S = encoding_indices.shape
    V, D = table.shape
    rows_per_w = B // NUM_WORKERS
    idx = encoding_indices.reshape(NUM_WORKERS, rows_per_w, S).astype(jnp.int32)
    mesh = plsc.VectorSubcoreMesh(core_axis_name="c", subcore_axis_name="s")
    out = pl.kernel(
        _gather_body,
        out_type=jax.ShapeDtypeStruct((B, S, D), jnp.float32),
        mesh=mesh,
        compiler_params=pltpu.CompilerParams(use_tc_tiling_on_sc=True),
        scratch_types=(
            [pltpu.VMEM((rows_per_w, S), jnp.int32)]
            + [pltpu.VMEM((CB, S, D), jnp.float32) for _ in range(NBUF)]
            + [pltpu.SemaphoreType.DMA for _ in range(2 * NBUF)]
        ),
    )(idx, table)
    return out


# speedup vs baseline: 3.1319x; 1.7408x over previous
import jax
import jax.numpy as jnp
from jax import lax
from jax.experimental import pallas as pl
from jax.experimental.pallas import tpu as pltpu
from jax.experimental.pallas import tpu_sc as plsc

NUM_CORES = 2
NUM_SUBCORES = 16
NUM_WORKERS = NUM_CORES * NUM_SUBCORES
CHUNK = 128
NBUF = 5


def _gather_body(idx_hbm, table_hbm, out_hbm, idx_v, *scratch):
    bufs = scratch[:NBUF]
    gsems = scratch[NBUF : 2 * NBUF]
    ssems = scratch[2 * NBUF :]
    wid = lax.axis_index("s") * NUM_CORES + lax.axis_index("c")
    nch = idx_hbm.shape[1]
    base = wid * (nch * CHUNK)
    nrounds = nch // NBUF

    pltpu.sync_copy(idx_hbm.at[wid], idx_v)

    def gather_start(b, c):
        pltpu.async_copy(table_hbm.at[idx_v.at[c]], bufs[b], gsems[b])

    def gather_wait(b, c):
        pltpu.make_async_copy(table_hbm.at[idx_v.at[c]], bufs[b], gsems[b]).wait()

    def out_slice(c):
        return out_hbm.at[pl.ds(base + c * CHUNK, CHUNK)]

    def store_start(b, c):
        pltpu.async_copy(bufs[b], out_slice(c), ssems[b])

    def store_wait(b, c):
        pltpu.make_async_copy(bufs[b], out_slice(c), ssems[b]).wait()

    for b in range(NBUF):
        gather_start(b, b)

    def round_body(r, carry):
        cbase = NBUF * r
        for b in range(NBUF):
            gather_wait(b, cbase + b)
            store_start(b, cbase + b)

        @pl.when(r < nrounds - 1)
        def _prefetch():
            for b in range(NBUF):
                store_wait(b, cbase + b)
                gather_start(b, cbase + NBUF + b)

        return carry

    lax.fori_loop(0, nrounds, round_body, 0)
    for b in range(NBUF):
        store_wait(b, nch - NBUF + b)


def kernel(encoding_indices, table):
    B, S = encoding_indices.shape
    V, D = table.shape
    n = B * S
    nch = n // (NUM_WORKERS * CHUNK)
    idx = encoding_indices.T.reshape(NUM_WORKERS, nch, CHUNK).astype(jnp.int32)
    mesh = plsc.VectorSubcoreMesh(core_axis_name="c", subcore_axis_name="s")
    out = pl.kernel(
        _gather_body,
        out_type=jax.ShapeDtypeStruct((n, D), jnp.float32),
        mesh=mesh,
        scratch_types=(
            [pltpu.VMEM((nch, CHUNK), jnp.int32)]
            + [pltpu.VMEM((CHUNK, D), jnp.float32) for _ in range(NBUF)]
            + [pltpu.SemaphoreType.DMA for _ in range(2 * NBUF)]
        ),
    )(idx, table)
    return out.reshape(S, B, D).transpose(1, 0, 2)


# speedup vs baseline: 3.1369x; 1.0016x over previous
import jax
import jax.numpy as jnp
from jax import lax
from jax.experimental import pallas as pl
from jax.experimental.pallas import tpu as pltpu
from jax.experimental.pallas import tpu_sc as plsc

NUM_CORES = 2
NUM_SUBCORES = 16
NUM_WORKERS = NUM_CORES * NUM_SUBCORES
CHUNK = 128
NBUF = 5


def _gather_body(idx_hbm, table_hbm, out_hbm, idx_v, *scratch):
    bufs = scratch[:NBUF]
    gsems = scratch[NBUF : 2 * NBUF]
    ssems = scratch[2 * NBUF :]
    wid = lax.axis_index("s") * NUM_CORES + lax.axis_index("c")
    nch = idx_hbm.shape[1]
    base = wid * (nch * CHUNK)
    nrounds = nch // NBUF

    pltpu.sync_copy(idx_hbm.at[wid], idx_v)

    def gather_start(b, c):
        pltpu.async_copy(table_hbm.at[idx_v.at[c]], bufs[b], gsems[b])

    def gather_wait(b, c):
        pltpu.make_async_copy(table_hbm.at[idx_v.at[c]], bufs[b], gsems[b]).wait()

    def out_slice(c):
        return out_hbm.at[pl.ds(base + c * CHUNK, CHUNK)]

    def store_start(b, c):
        pltpu.async_copy(bufs[b], out_slice(c), ssems[b])

    def store_wait(b, c):
        pltpu.make_async_copy(bufs[b], out_slice(c), ssems[b]).wait()

    for b in range(NBUF):
        gather_start(b, b)

    def round_body(r, carry):
        cbase = NBUF * r
        for b in range(NBUF):
            gather_wait(b, cbase + b)
            store_start(b, cbase + b)

        @pl.when(r < nrounds - 1)
        def _prefetch():
            for b in range(NBUF):
                store_wait(b, cbase + b)
                gather_start(b, cbase + NBUF + b)

        return carry

    lax.fori_loop(0, nrounds, round_body, 0)
    for b in range(NBUF):
        store_wait(b, nch - NBUF + b)


def kernel(encoding_indices, table):
    B, S = encoding_indices.shape
    V, D = table.shape
    n = B * S
    nch = n // (NUM_WORKERS * CHUNK)
    idx = encoding_indices.T.reshape(NUM_WORKERS, nch, CHUNK).astype(jnp.int32)
    mesh = plsc.VectorSubcoreMesh(core_axis_name="c", subcore_axis_name="s")
    out = pl.kernel(
        _gather_body,
        out_type=jax.ShapeDtypeStruct((n, D), jnp.float32),
        mesh=mesh,
        compiler_params=pltpu.CompilerParams(use_tc_tiling_on_sc=False),
        scratch_types=(
            [pltpu.VMEM((nch, CHUNK), jnp.int32)]
            + [pltpu.VMEM((CHUNK, D), jnp.float32) for _ in range(NBUF)]
            + [pltpu.SemaphoreType.DMA for _ in range(2 * NBUF)]
        ),
    )(idx, table)
    return out.reshape(S, B, D).transpose(1, 0, 2)
